# baseline (speedup 1.0000x reference)
import functools

import jax
import jax.numpy as jnp
from jax.experimental import pallas as pl
from jax.experimental.pallas import tpu as pltpu

_PROXIES = 16


def _fused_kernel(d_ref, wa_ref, wb_ref, wc_ref, wd_ref, o_ref, db_ref, *, bm, bn):
    @pl.when(pl.program_id(0) == 0)
    def _():
        d = d_ref[...]
        norm = jnp.sqrt(jnp.sum(d * d, axis=1, keepdims=True))
        db_ref[...] = (d * (1.0 / jnp.maximum(norm, 1e-12))).astype(jnp.bfloat16)

    db = db_ref[...]
    qc = (bm // 4) // _PROXIES

    def score(w_q_ref):
        x = jax.lax.dot_general(
            w_q_ref[...].astype(jnp.bfloat16), db,
            (((1,), (1,)), ((), ())),
            preferred_element_type=jnp.float32,
        )
        return jnp.max(x.reshape(qc, _PROXIES, bn), axis=1)

    for k, wr in enumerate((wa_ref, wb_ref, wc_ref, wd_ref)):
        o_ref[k * qc:(k + 1) * qc, :] = score(wr)


def kernel(data, w1):
    batch, embed = data.shape
    rows = w1.shape[0]
    num_classes = rows // _PROXIES

    bm = 4096
    bn = batch
    grid = (rows // bm,)

    fn = functools.partial(_fused_kernel, bm=bm, bn=bn)
    out = pl.pallas_call(
        fn,
        grid=grid,
        in_specs=[
            pl.BlockSpec((bn, embed), lambda i: (0, 0)),
        ] + [
            pl.BlockSpec((bm // 4, embed),
                         functools.partial(lambda k, i: (4 * i + k, 0), k))
            for k in range(4)
        ],
        out_specs=pl.BlockSpec((bm // _PROXIES, bn), lambda i: (i, 0)),
        out_shape=jax.ShapeDtypeStruct((num_classes, batch), jnp.float32),
        scratch_shapes=[pltpu.VMEM((bn, embed), jnp.bfloat16)],
    )(data, *([w1] * 4))
    return out


# speedup vs baseline: 1.0264x; 1.0264x over previous
import functools

import jax
import jax.numpy as jnp
from jax.experimental import pallas as pl
from jax.experimental.pallas import tpu as pltpu

_PROXIES = 16


def _fused_kernel(d_ref, wa_ref, wb_ref, o_ref, db_ref, *, bm, bn):
    @pl.when(pl.program_id(0) == 0)
    def _():
        d = d_ref[...]
        norm = jnp.sqrt(jnp.sum(d * d, axis=1, keepdims=True))
        db_ref[...] = (d * (1.0 / jnp.maximum(norm, 1e-12))).astype(jnp.bfloat16)

    db = db_ref[...]
    hc = (bm // 2) // _PROXIES

    def score(w_half_ref):
        x = jax.lax.dot_general(
            w_half_ref[...].astype(jnp.bfloat16), db,
            (((1,), (1,)), ((), ())),
            preferred_element_type=jnp.float32,
        )
        return jnp.max(x.reshape(hc, _PROXIES, bn), axis=1)

    o_ref[:hc, :] = score(wa_ref)
    o_ref[hc:, :] = score(wb_ref)


def kernel(data, w1):
    batch, embed = data.shape
    rows = w1.shape[0]
    num_classes = rows // _PROXIES

    bm = 2048
    bn = batch
    grid = (rows // bm,)

    fn = functools.partial(_fused_kernel, bm=bm, bn=bn)
    out = pl.pallas_call(
        fn,
        grid=grid,
        in_specs=[
            pl.BlockSpec((bn, embed), lambda i: (0, 0)),
            pl.BlockSpec((bm // 2, embed), lambda i: (2 * i, 0)),
            pl.BlockSpec((bm // 2, embed), lambda i: (2 * i + 1, 0)),
        ],
        out_specs=pl.BlockSpec((bm // _PROXIES, bn), lambda i: (i, 0)),
        out_shape=jax.ShapeDtypeStruct((num_classes, batch), jnp.float32),
        scratch_shapes=[pltpu.VMEM((bn, embed), jnp.bfloat16)],
    )(data, w1, w1)
    return out
